# baseline (speedup 1.0000x reference)
import functools

import jax
import jax.numpy as jnp
from jax.experimental import pallas as pl

IOU_THR = 0.4
BI = 512
BJ = 512


def _nms_decay_body(x1i_ref, y1i_ref, x2i_ref, y2i_ref, si_ref,
                    x1j_ref, y1j_ref, x2j_ref, y2j_ref, sj_ref,
                    out_ref):
    x1i = x1i_ref[...]
    y1i = y1i_ref[...]
    x2i = x2i_ref[...]
    y2i = y2i_ref[...]
    si = si_ref[...]
    area_i = (x2i - x1i + 1.0) * (y2i - y1i + 1.0)

    nchunks = x1j_ref.shape[1] // BJ

    def body(c, acc):
        sl = pl.ds(c * BJ, BJ)
        x1j = x1j_ref[:, sl]
        y1j = y1j_ref[:, sl]
        x2j = x2j_ref[:, sl]
        y2j = y2j_ref[:, sl]
        sj = sj_ref[:, sl]
        area_j = (x2j - x1j + 1.0) * (y2j - y1j + 1.0)

        w = jnp.maximum(jnp.minimum(x2i, x2j) - jnp.maximum(x1i, x1j) + 1.0, 0.0)
        h = jnp.maximum(jnp.minimum(y2i, y2j) - jnp.maximum(y1i, y1j) + 1.0, 0.0)
        inter = w * h
        union = (area_i + area_j) - inter
        iou = inter / union
        cond = jnp.logical_and(iou > IOU_THR, sj > si)
        f = jnp.where(cond, 1.0 - iou, 1.0)
        return acc * f

    acc = jax.lax.fori_loop(0, nchunks, body,
                            jnp.ones((BI, BJ), jnp.float32))

    width = BJ
    while width > 1:
        width //= 2
        acc = acc[:, :width] * acc[:, width:2 * width]

    out_ref[...] = si * acc


@jax.jit
def kernel(boxes, scores):
    n = boxes.shape[0]
    npad = ((n + BI - 1) // BI) * BI
    pad = npad - n

    x1 = jnp.pad(boxes[:, 0], (0, pad))
    y1 = jnp.pad(boxes[:, 1], (0, pad))
    x2 = jnp.pad(boxes[:, 2], (0, pad))
    y2 = jnp.pad(boxes[:, 3], (0, pad))
    s = jnp.pad(scores, (0, pad), constant_values=-1e30)

    col = lambda a: a.reshape(npad, 1)
    row = lambda a: a.reshape(1, npad)

    grid = (npad // BI,)
    ispec = pl.BlockSpec((BI, 1), lambda i: (i, 0))
    jspec = pl.BlockSpec((1, npad), lambda i: (0, 0))

    out = pl.pallas_call(
        _nms_decay_body,
        grid=grid,
        in_specs=[ispec, ispec, ispec, ispec, ispec,
                  jspec, jspec, jspec, jspec, jspec],
        out_specs=pl.BlockSpec((BI, 1), lambda i: (i, 0)),
        out_shape=jax.ShapeDtypeStruct((npad, 1), jnp.float32),
    )(col(x1), col(y1), col(x2), col(y2), col(s),
      row(x1), row(y1), row(x2), row(y2), row(s))

    return out[:n, 0]
